# baseline (speedup 1.0000x reference)
import functools

import jax
import jax.numpy as jnp
from jax import lax
from jax.experimental import pallas as pl
from jax.experimental.pallas import tpu as pltpu
from jax.experimental.pallas import tpu_sc as plsc

N_NODES = 10000
N_EDGES = 320000
D_FEAT = 128
N_OUT = 4

NC, NS, L = 2, 16, 16
NW = NC * NS
BLK = 128
N_BLKS = N_EDGES // BLK
BASE_BPW = N_BLKS // NW
EXTRA = N_BLKS - BASE_BPW * NW
MAX_BPW = BASE_BPW + 1
GPB = BLK // L


def _mm_body(w_ref, x_ref, yt_ref):
    yt_ref[0] = jax.lax.dot_general(
        w_ref[...], x_ref[...], (((1,), (1,)), ((), ())),
        preferred_element_type=jnp.float32)


_MM_GRID = 10
_MM_ROWS = N_NODES // _MM_GRID


def _node_proj_t(w, x):
    return pl.pallas_call(
        _mm_body,
        grid=(_MM_GRID,),
        in_specs=[
            pl.BlockSpec((N_OUT, D_FEAT), lambda i: (0, 0)),
            pl.BlockSpec((_MM_ROWS, D_FEAT), lambda i: (i, 0)),
        ],
        out_specs=pl.BlockSpec((1, N_OUT, _MM_ROWS), lambda i: (i, 0, 0)),
        out_shape=jax.ShapeDtypeStruct((_MM_GRID, N_OUT, _MM_ROWS),
                                       jnp.float32),
    )(w, x)


@functools.cache
def _make_edge_kernel():
    mesh = plsc.VectorSubcoreMesh(core_axis_name="c", subcore_axis_name="s")

    @functools.partial(
        pl.kernel,
        mesh=mesh,
        out_type=jax.ShapeDtypeStruct((2, N_BLKS, 2, BLK), jnp.float32),
        scratch_types=[
            pltpu.VMEM((N_NODES * N_OUT,), jnp.float32),
            pltpu.VMEM((MAX_BPW, 2, BLK), jnp.int32),
            pltpu.VMEM((2, MAX_BPW, 2, BLK), jnp.float32),
        ],
        compiler_params=pltpu.CompilerParams(
            needs_layout_passes=False, use_tc_tiling_on_sc=False),
    )
    def edge_kernel(y_hbm, ei_hbm, out_hbm, y_v, ei_v, out_v):
        wid = lax.axis_index("s") * NC + lax.axis_index("c")
        nb = jnp.where(wid < EXTRA, MAX_BPW, BASE_BPW)
        b0 = wid * BASE_BPW + jnp.minimum(wid, EXTRA)
        pltpu.sync_copy(y_hbm, y_v)
        pltpu.sync_copy(ei_hbm.at[pl.ds(b0, nb)], ei_v.at[pl.ds(0, nb)])

        @plsc.parallel_loop(0, nb * GPB)
        def body(g):
            blk = g // GPB
            el0 = (g % GPB) * L
            rv = ei_v[blk, 0, pl.ds(el0, L)]
            cv = ei_v[blk, 1, pl.ds(el0, L)]
            for j in range(N_OUT):
                a = plsc.load_gather(y_v, [rv + (j * N_NODES)])
                b = plsc.load_gather(y_v, [cv + (j * N_NODES)])
                e2 = jnp.exp((a + b) * 2.0)
                t = 1.0 - 2.0 / (e2 + 1.0)
                out_v[j // 2, blk, j % 2, pl.ds(el0, L)] = t

        for i in range(2):
            for jj in range(2):
                pltpu.sync_copy(out_v.at[i, pl.ds(0, nb), jj],
                                out_hbm.at[i, pl.ds(b0, nb), jj])

    return edge_kernel


def kernel(x, edge_index, W):
    yt = _node_proj_t(W, x).transpose(1, 0, 2).reshape(N_OUT, N_NODES)
    ei_blocks = edge_index.reshape(2, N_BLKS, BLK).transpose(1, 0, 2)
    out = _make_edge_kernel()(yt.reshape(-1), ei_blocks)
    return out.transpose(1, 3, 0, 2).reshape(N_EDGES, 2, 2)


# speedup vs baseline: 1.1197x; 1.1197x over previous
import functools

import jax
import jax.numpy as jnp
from jax import lax
from jax.experimental import pallas as pl
from jax.experimental.pallas import tpu as pltpu
from jax.experimental.pallas import tpu_sc as plsc

N_NODES = 10000
N_EDGES = 320000
D_FEAT = 128
N_OUT = 4

NC, NS, L = 2, 16, 16
NW = NC * NS
BLK = 128
N_BLKS = N_EDGES // BLK
BASE_BPW = N_BLKS // NW
EXTRA = N_BLKS - BASE_BPW * NW
MAX_BPW = BASE_BPW + 1
GPB = BLK // L


def _mm_body(w_ref, x_ref, yt_ref):
    yt_ref[...] = jax.lax.dot_general(
        w_ref[...], x_ref[...], (((1,), (1,)), ((), ())),
        preferred_element_type=jnp.float32)


def _node_proj_t(w, x):
    return pl.pallas_call(
        _mm_body,
        out_shape=jax.ShapeDtypeStruct((N_OUT, N_NODES), jnp.float32),
    )(w, x)


@functools.cache
def _make_edge_kernel():
    mesh = plsc.VectorSubcoreMesh(core_axis_name="c", subcore_axis_name="s")

    @functools.partial(
        pl.kernel,
        mesh=mesh,
        out_type=jax.ShapeDtypeStruct((2, N_BLKS, 2, BLK), jnp.float32),
        scratch_types=[
            pltpu.VMEM((N_NODES * N_OUT,), jnp.float32),
            pltpu.VMEM((MAX_BPW, 2, BLK), jnp.int32),
            pltpu.VMEM((2, MAX_BPW, 2, BLK), jnp.float32),
            pltpu.SemaphoreType.DMA,
            pltpu.SemaphoreType.DMA,
        ],
        compiler_params=pltpu.CompilerParams(
            needs_layout_passes=False, use_tc_tiling_on_sc=False),
    )
    def edge_kernel(y_hbm, ei_hbm, out_hbm, y_v, ei_v, out_v,
                    sem_in, sem_out):
        wid = lax.axis_index("s") * NC + lax.axis_index("c")
        nb = jnp.where(wid < EXTRA, MAX_BPW, BASE_BPW)
        b0 = wid * BASE_BPW + jnp.minimum(wid, EXTRA)
        cy = pltpu.async_copy(y_hbm, y_v, sem_in)
        ce = pltpu.async_copy(ei_hbm.at[pl.ds(b0, nb)],
                              ei_v.at[pl.ds(0, nb)], sem_in)
        cy.wait()
        ce.wait()

        def run_groups(g_lo, g_hi):
            @plsc.parallel_loop(g_lo, g_hi)
            def body(g):
                blk = g // GPB
                el0 = (g % GPB) * L
                rv = ei_v[blk, 0, pl.ds(el0, L)]
                cv = ei_v[blk, 1, pl.ds(el0, L)]
                for j in range(N_OUT):
                    a = plsc.load_gather(y_v, [rv + (j * N_NODES)])
                    b = plsc.load_gather(y_v, [cv + (j * N_NODES)])
                    e2 = jnp.exp((a + b) * 2.0)
                    t = 1.0 - 2.0 / (e2 + 1.0)
                    out_v[j // 2, blk, j % 2, pl.ds(el0, L)] = t

        def flush_blocks(lo, n):
            return [
                pltpu.async_copy(out_v.at[i, pl.ds(lo, n), jj],
                                 out_hbm.at[i, pl.ds(b0 + lo, n), jj],
                                 sem_out)
                for i in range(2) for jj in range(2)
            ]

        h1 = nb // 2
        run_groups(0, h1 * GPB)
        c1 = flush_blocks(0, h1)
        run_groups(h1 * GPB, nb * GPB)
        c2 = flush_blocks(h1, nb - h1)
        for c in c1 + c2:
            c.wait()

    return edge_kernel


def kernel(x, edge_index, W):
    yt = _node_proj_t(W, x)
    ei_blocks = edge_index.reshape(2, N_BLKS, BLK).transpose(1, 0, 2)
    out = _make_edge_kernel()(yt.reshape(-1), ei_blocks)
    return out.transpose(1, 3, 0, 2).reshape(N_EDGES, 2, 2)
